# Initial kernel scaffold; baseline (speedup 1.0000x reference)
import jax
import jax.numpy as jnp
from jax.experimental import pallas as pl
from jax.experimental.pallas import tpu as pltpu

N, C, H, W = 32, 256, 56, 56
G = 16
EPS = 1e-05
N_ITER = 10
HW = H * W
R = N * C
BR = 256
NBLK = R // BR
CORES = 2
INNER = NBLK // CORES
M_TOT = N * (C // G) * HW


def _stats_kernel(x_ref, q_ref, s_ref):
    i = pl.program_id(1)

    @pl.when(i == 0)
    def _():
        q_ref[...] = jnp.zeros_like(q_ref)
        s_ref[...] = jnp.zeros_like(s_ref)

    a = x_ref[...]
    q = jax.lax.dot_general(a, a, (((1,), (1,)), ((), ())),
                            preferred_element_type=jnp.float32)
    q_ref[0] += q
    s_ref[0] += jnp.sum(a, axis=1, keepdims=True)


def _finish_kernel(q_ref, s_ref, w_ref, b_ref, d_ref, o_ref):
    Q = q_ref[0] + q_ref[1]
    s = s_ref[0] + s_ref[1]

    ri = jax.lax.broadcasted_iota(jnp.int32, (BR, BR), 0)
    ci = jax.lax.broadcasted_iota(jnp.int32, (BR, BR), 1)
    bd = ((ri // G) == (ci // G)).astype(jnp.float32)
    gi = jax.lax.broadcasted_iota(jnp.int32, (G, BR), 0)
    cg = jax.lax.broadcasted_iota(jnp.int32, (G, BR), 1)
    sel = ((cg % G) == gi).astype(jnp.float32)
    r2 = jax.lax.broadcasted_iota(jnp.int32, (BR, G), 0)
    g2 = jax.lax.broadcasted_iota(jnp.int32, (BR, G), 1)
    sel_t = ((r2 % G) == g2).astype(jnp.float32)
    eye = (jax.lax.broadcasted_iota(jnp.int32, (G, G), 0)
           == jax.lax.broadcasted_iota(jnp.int32, (G, G), 1)
           ).astype(jnp.float32)

    inv_m = jnp.float32(1.0 / M_TOT)
    sg = jnp.dot(sel, s, preferred_element_type=jnp.float32)
    mean = sg * inv_m
    sg_row = jax.lax.dot_general(s, sel_t, (((0,), (0,)), ((), ())),
                                 preferred_element_type=jnp.float32)
    mean_row = sg_row * inv_m
    p16 = jnp.dot(jnp.dot(sel, Q * bd, preferred_element_type=jnp.float32),
                  sel_t, preferred_element_type=jnp.float32)
    cov = p16 * inv_m - mean * mean_row + EPS * eye

    norm_a = jnp.sqrt(jnp.sum(cov * cov))
    y = cov / norm_a
    z = eye
    for _ in range(N_ITER):
        t = 0.5 * (3.0 * eye - jnp.dot(z, y, preferred_element_type=jnp.float32))
        y = jnp.dot(y, t, preferred_element_type=jnp.float32)
        z = jnp.dot(t, z, preferred_element_type=jnp.float32)
    decorr = z / jnp.sqrt(norm_a)

    w = w_ref[...]
    b = b_ref[...]
    d_tile = jnp.dot(sel_t, jnp.dot(decorr, sel, preferred_element_type=jnp.float32),
                     preferred_element_type=jnp.float32)
    d_ref[...] = d_tile * bd * w
    dm = jnp.dot(decorr, mean, preferred_element_type=jnp.float32)
    dmt = jnp.dot(sel_t, dm, preferred_element_type=jnp.float32)
    o_ref[...] = b - w * dmt


def _apply_kernel(x_ref, d_ref, o_ref, y_ref):
    y_ref[...] = (jnp.dot(d_ref[...], x_ref[...],
                          preferred_element_type=jnp.float32)
                  + o_ref[...])


def kernel(x, weight, bias):
    x2d = x.reshape(R, HW)
    w = weight.reshape(C, 1)
    b = bias.reshape(C, 1)

    qp, sp = pl.pallas_call(
        _stats_kernel,
        grid=(CORES, INNER),
        in_specs=[pl.BlockSpec((BR, HW), lambda p, i: (p * INNER + i, 0))],
        out_specs=[
            pl.BlockSpec((1, BR, BR), lambda p, i: (p, 0, 0)),
            pl.BlockSpec((1, BR, 1), lambda p, i: (p, 0, 0)),
        ],
        out_shape=[
            jax.ShapeDtypeStruct((CORES, BR, BR), jnp.float32),
            jax.ShapeDtypeStruct((CORES, BR, 1), jnp.float32),
        ],
        compiler_params=pltpu.CompilerParams(
            dimension_semantics=("core_parallel", "arbitrary"),
        ),
        name="decorr_stats",
    )(x2d)

    dbig, off = pl.pallas_call(
        _finish_kernel,
        out_shape=[
            jax.ShapeDtypeStruct((BR, BR), jnp.float32),
            jax.ShapeDtypeStruct((BR, 1), jnp.float32),
        ],
        name="decorr_finish",
    )(qp, sp, w, b)

    y2d = pl.pallas_call(
        _apply_kernel,
        grid=(CORES, INNER),
        in_specs=[
            pl.BlockSpec((BR, HW), lambda p, i: (p * INNER + i, 0)),
            pl.BlockSpec((BR, BR), lambda p, i: (0, 0)),
            pl.BlockSpec((BR, 1), lambda p, i: (0, 0)),
        ],
        out_specs=pl.BlockSpec((BR, HW), lambda p, i: (p * INNER + i, 0)),
        out_shape=jax.ShapeDtypeStruct((R, HW), jnp.float32),
        compiler_params=pltpu.CompilerParams(
            dimension_semantics=("core_parallel", "arbitrary"),
        ),
        name="decorr_apply",
    )(x2d, dbig, off)

    return y2d.reshape(N, C, H, W)


# speedup vs baseline: 8.5774x; 8.5774x over previous
import jax
import jax.numpy as jnp
from jax.experimental import pallas as pl
from jax.experimental.pallas import tpu as pltpu

N, C, H, W = 32, 256, 56, 56
G = 16
EPS = 1e-05
N_ITER = 10
HW = H * W
R = N * C
BR = 256
NBLK = R // BR
CORES = 2
INNER = NBLK // CORES
M_TOT = N * (C // G) * HW


def _stats_kernel(x_ref, q_ref, s_ref):
    i = pl.program_id(1)

    @pl.when(i == 0)
    def _():
        q_ref[...] = jnp.zeros_like(q_ref)
        s_ref[...] = jnp.zeros_like(s_ref)

    a = x_ref[...]
    q = jax.lax.dot_general(a, a, (((1,), (1,)), ((), ())),
                            preferred_element_type=jnp.float32)
    q_ref[0] += q
    s_ref[0] += jnp.sum(a, axis=1, keepdims=True)


def _finish_kernel(q_ref, s_ref, w_ref, b_ref, d_ref, o_ref):
    Q = q_ref[0] + q_ref[1]
    s = s_ref[0] + s_ref[1]

    ri = jax.lax.broadcasted_iota(jnp.int32, (BR, BR), 0)
    ci = jax.lax.broadcasted_iota(jnp.int32, (BR, BR), 1)
    bd = ((ri // G) == (ci // G)).astype(jnp.float32)
    gi = jax.lax.broadcasted_iota(jnp.int32, (G, BR), 0)
    cg = jax.lax.broadcasted_iota(jnp.int32, (G, BR), 1)
    sel = ((cg % G) == gi).astype(jnp.float32)
    r2 = jax.lax.broadcasted_iota(jnp.int32, (BR, G), 0)
    g2 = jax.lax.broadcasted_iota(jnp.int32, (BR, G), 1)
    sel_t = ((r2 % G) == g2).astype(jnp.float32)
    eye = (jax.lax.broadcasted_iota(jnp.int32, (G, G), 0)
           == jax.lax.broadcasted_iota(jnp.int32, (G, G), 1)
           ).astype(jnp.float32)

    inv_m = jnp.float32(1.0 / M_TOT)
    sg = jnp.dot(sel, s, preferred_element_type=jnp.float32)
    mean = sg * inv_m
    sg_row = jax.lax.dot_general(s, sel_t, (((0,), (0,)), ((), ())),
                                 preferred_element_type=jnp.float32)
    mean_row = sg_row * inv_m
    p16 = jnp.dot(jnp.dot(sel, Q * bd, preferred_element_type=jnp.float32),
                  sel_t, preferred_element_type=jnp.float32)
    cov = p16 * inv_m - mean * mean_row + EPS * eye

    norm_a = jnp.sqrt(jnp.sum(cov * cov))
    y = cov / norm_a
    z = eye
    for _ in range(N_ITER):
        t = 0.5 * (3.0 * eye - jnp.dot(z, y, preferred_element_type=jnp.float32))
        y = jnp.dot(y, t, preferred_element_type=jnp.float32)
        z = jnp.dot(t, z, preferred_element_type=jnp.float32)
    decorr = z / jnp.sqrt(norm_a)

    w = w_ref[...]
    b = b_ref[...]
    d_tile = jnp.dot(sel_t, jnp.dot(decorr, sel, preferred_element_type=jnp.float32),
                     preferred_element_type=jnp.float32)
    d_ref[...] = d_tile * bd * w
    dm = jnp.dot(decorr, mean, preferred_element_type=jnp.float32)
    dmt = jnp.dot(sel_t, dm, preferred_element_type=jnp.float32)
    o_ref[...] = b - w * dmt


def _apply_kernel(x_ref, d_ref, o_ref, y_ref):
    y_ref[...] = (jnp.dot(d_ref[...], x_ref[...],
                          preferred_element_type=jnp.float32)
                  + o_ref[...])


def kernel(x, weight, bias):
    x2d = x.reshape(R, HW)
    w = weight.reshape(C, 1)
    b = bias.reshape(C, 1)

    qp, sp = pl.pallas_call(
        _stats_kernel,
        grid=(CORES, INNER),
        in_specs=[pl.BlockSpec((BR, HW), lambda p, i: (p * INNER + i, 0))],
        out_specs=[
            pl.BlockSpec((1, BR, BR), lambda p, i: (p, 0, 0)),
            pl.BlockSpec((1, BR, 1), lambda p, i: (p, 0, 0)),
        ],
        out_shape=[
            jax.ShapeDtypeStruct((CORES, BR, BR), jnp.float32),
            jax.ShapeDtypeStruct((CORES, BR, 1), jnp.float32),
        ],
        compiler_params=pltpu.CompilerParams(
            dimension_semantics=("parallel", "arbitrary"),
        ),
        name="decorr_stats",
    )(x2d)

    dbig, off = pl.pallas_call(
        _finish_kernel,
        out_shape=[
            jax.ShapeDtypeStruct((BR, BR), jnp.float32),
            jax.ShapeDtypeStruct((BR, 1), jnp.float32),
        ],
        name="decorr_finish",
    )(qp, sp, w, b)

    y2d = pl.pallas_call(
        _apply_kernel,
        grid=(CORES, INNER),
        in_specs=[
            pl.BlockSpec((BR, HW), lambda p, i: (p * INNER + i, 0)),
            pl.BlockSpec((BR, BR), lambda p, i: (0, 0)),
            pl.BlockSpec((BR, 1), lambda p, i: (0, 0)),
        ],
        out_specs=pl.BlockSpec((BR, HW), lambda p, i: (p * INNER + i, 0)),
        out_shape=jax.ShapeDtypeStruct((R, HW), jnp.float32),
        compiler_params=pltpu.CompilerParams(
            dimension_semantics=("parallel", "arbitrary"),
        ),
        name="decorr_apply",
    )(x2d, dbig, off)

    return y2d.reshape(N, C, H, W)
